# baseline (speedup 1.0000x reference)
import functools

import jax
import jax.numpy as jnp
from jax import lax
from jax.experimental import pallas as pl
from jax.experimental.pallas import tpu as pltpu
from jax.experimental.pallas import tpu_sc as plsc

_N_GAUSS = 2_000_000
_N_FACES = 1_000_000
_B = 128
_NCHUNKS = _N_GAUSS // _B
_NC = 2
_NS = 16
_NW = _NC * _NS
_L = 16


def _splat(v):
    return jnp.full((_L,), v, jnp.int32)


def _body(fid_hbm, face_hbm, verts_hbm, bary_hbm,
          out_hbm,
          fid_v, fidx_v, vid_v, pidx_v, g_v, bary_v, out_v,
          s_fid, s_bary, s_vid, s_pl, s_out):
    wid = lax.axis_index("s") * _NC + lax.axis_index("c")
    n_w = (_NCHUNKS - wid + _NW - 1) // _NW
    lane = lax.iota(jnp.int32, _L)

    def chunk(i):
        return wid + i * _NW

    def fid_copy(i):
        base = pl.multiple_of(chunk(i) * _B, _B)
        return pltpu.make_async_copy(
            fid_hbm.at[pl.ds(base, _B)], fid_v.at[i % 2], s_fid)

    def bary_copy(i):
        base3 = pl.multiple_of(chunk(i) * (3 * _B), 3 * _B)
        return pltpu.make_async_copy(
            bary_hbm.at[pl.ds(base3, 3 * _B)], bary_v.at[i % 2], s_bary)

    def build_fidx(i):
        p = _splat(i % 2)
        for t in range(_B // _L):
            idx = lane + (t * _L)
            f = plsc.load_gather(fid_v, [p, idx])
            for j in range(3):
                row = _splat((i % 2) * 3 + j)
                plsc.store_scatter(fidx_v, [row, idx], f + (j * _N_FACES))

    def vid_copies(i):
        return [pltpu.make_async_copy(
                    face_hbm.at[fidx_v.at[(i % 2) * 3 + j]],
                    vid_v.at[(i % 3) * 3 + j], s_vid)
                for j in range(3)]

    def build_pidx(i):
        for j in range(3):
            vrow = _splat((i % 3) * 3 + j)
            for t in range(_B // _L):
                idx = lane + (t * _L)
                v3 = plsc.load_gather(vid_v, [vrow, idx]) * 3
                for c in range(3):
                    row = _splat((i % 2) * 9 + 3 * j + c)
                    plsc.store_scatter(pidx_v, [row, idx], v3 + c)

    def plane_copies(i):
        return [pltpu.make_async_copy(
                    verts_hbm.at[pidx_v.at[(i % 2) * 9 + 3 * j + c]],
                    g_v.at[(i % 2) * 9 + 3 * j + c], s_pl)
                for j in range(3) for c in range(3)]

    def out_copy(i):
        base3 = pl.multiple_of(chunk(i) * (3 * _B), 3 * _B)
        return pltpu.make_async_copy(
            out_v.at[i % 2], out_hbm.at[pl.ds(base3, 3 * _B)], s_out)

    def start(cps):
        if not isinstance(cps, (list, tuple)):
            cps = [cps]
        for cp in cps:
            cp.start()

    def wait(cps):
        if not isinstance(cps, (list, tuple)):
            cps = [cps]
        for cp in cps:
            cp.wait()

    start(fid_copy(0))
    start(fid_copy(1))
    start(bary_copy(0))
    wait(fid_copy(0))
    build_fidx(0)
    start(vid_copies(0))
    wait(vid_copies(0))
    build_pidx(0)
    start(plane_copies(0))
    wait(fid_copy(1))
    build_fidx(1)
    start(vid_copies(1))
    start(fid_copy(2))

    def loop_body(i, carry):
        p = i % 2

        @pl.when(i + 1 < n_w)
        def _():
            wait(vid_copies(i + 1))
            build_pidx(i + 1)
            start(plane_copies(i + 1))
            start(bary_copy(i + 1))

        @pl.when(i + 2 < n_w)
        def _():
            wait(fid_copy(i + 2))
            build_fidx(i + 2)
            start(vid_copies(i + 2))

        @pl.when(i + 3 < n_w)
        def _():
            start(fid_copy(i + 3))

        wait(plane_copies(i))
        wait(bary_copy(i))

        @pl.when(i >= 2)
        def _():
            wait(out_copy(i - 2))

        pvec = _splat(p)
        gbase = p * 9
        for t in range(_B // _L):
            idx = lane + (t * _L)
            idx3 = idx * 3
            w0 = plsc.load_gather(bary_v, [pvec, idx3])
            w1 = plsc.load_gather(bary_v, [pvec, idx3 + 1])
            w2 = plsc.load_gather(bary_v, [pvec, idx3 + 2])
            for c in range(3):
                g0 = plsc.load_gather(g_v, [_splat(gbase + c), idx])
                g1 = plsc.load_gather(g_v, [_splat(gbase + 3 + c), idx])
                g2 = plsc.load_gather(g_v, [_splat(gbase + 6 + c), idx])
                acc = w0 * g0 + w1 * g1 + w2 * g2
                plsc.store_scatter(out_v, [pvec, idx3 + c], acc)
        start(out_copy(i))
        return carry

    lax.fori_loop(0, n_w, loop_body, 0)
    wait(out_copy(n_w - 2))
    wait(out_copy(n_w - 1))


@jax.jit
def _sc_interp(fid, face_flat, verts_flat, bary_flat):
    mesh = plsc.VectorSubcoreMesh(core_axis_name="c", subcore_axis_name="s")
    run = functools.partial(
        pl.kernel,
        mesh=mesh,
        compiler_params=pltpu.CompilerParams(needs_layout_passes=False),
        out_type=jax.ShapeDtypeStruct((3 * _N_GAUSS,), jnp.float32),
        scratch_types=[
            pltpu.VMEM((2, _B), jnp.int32),
            pltpu.VMEM((6, _B), jnp.int32),
            pltpu.VMEM((9, _B), jnp.int32),
            pltpu.VMEM((18, _B), jnp.int32),
            pltpu.VMEM((18, _B), jnp.float32),
            pltpu.VMEM((2, 3 * _B), jnp.float32),
            pltpu.VMEM((2, 3 * _B), jnp.float32),
            pltpu.SemaphoreType.DMA,
            pltpu.SemaphoreType.DMA,
            pltpu.SemaphoreType.DMA,
            pltpu.SemaphoreType.DMA,
            pltpu.SemaphoreType.DMA,
        ],
    )(_body)
    return run(fid, face_flat, verts_flat, bary_flat)


def kernel(deformed_vertices, face, face_ids, face_bary):
    face_flat = face.reshape(-1)
    verts_flat = deformed_vertices.reshape(-1)
    bary_flat = face_bary.reshape(-1)
    out_flat = _sc_interp(face_ids, face_flat, verts_flat, bary_flat)
    return out_flat.reshape(_N_GAUSS, 3)


# speedup vs baseline: 7.1815x; 7.1815x over previous
import functools

import jax
import jax.numpy as jnp
from jax import lax
from jax.experimental import pallas as pl
from jax.experimental.pallas import tpu as pltpu
from jax.experimental.pallas import tpu_sc as plsc

_N_GAUSS = 2_000_000
_B = 128
_NCHUNKS = _N_GAUSS // _B
_NC = 2
_NS = 16
_NW = _NC * _NS
_L = 16


def _splat(v):
    return jnp.full((_L,), v, jnp.int32)


def _body(fid_hbm, f0_hbm, f1_hbm, f2_hbm, vx_hbm, vy_hbm, vz_hbm,
          b0_hbm, b1_hbm, b2_hbm,
          ox_hbm, oy_hbm, oz_hbm,
          fid_v, vid_v, g_v, bary_v, out_v,
          s_fid, s_bary, s_vid, s_pl, s_out):
    wid = lax.axis_index("s") * _NC + lax.axis_index("c")
    n_w = (_NCHUNKS - wid + _NW - 1) // _NW
    lane = lax.iota(jnp.int32, _L)
    planes = (vx_hbm, vy_hbm, vz_hbm)
    faces = (f0_hbm, f1_hbm, f2_hbm)
    barys = (b0_hbm, b1_hbm, b2_hbm)
    outs = (ox_hbm, oy_hbm, oz_hbm)

    def chunk(i):
        return wid + i * _NW

    def fid_copy(i):
        base = pl.multiple_of(chunk(i) * _B, _B)
        return pltpu.make_async_copy(
            fid_hbm.at[pl.ds(base, _B)], fid_v.at[i % 2], s_fid)

    def bary_copies(i):
        base = pl.multiple_of(chunk(i) * _B, _B)
        return [pltpu.make_async_copy(
                    barys[j].at[pl.ds(base, _B)],
                    bary_v.at[(i % 2) * 3 + j], s_bary)
                for j in range(3)]

    def vid_copies(i):
        return [pltpu.make_async_copy(
                    faces[j].at[fid_v.at[i % 2]], vid_v.at[(i % 3) * 3 + j],
                    s_vid)
                for j in range(3)]

    def plane_copies(i):
        return [pltpu.make_async_copy(
                    planes[c].at[vid_v.at[(i % 3) * 3 + j]],
                    g_v.at[(i % 2) * 9 + 3 * j + c], s_pl)
                for j in range(3) for c in range(3)]

    def out_copies(i):
        base = pl.multiple_of(chunk(i) * _B, _B)
        return [pltpu.make_async_copy(
                    out_v.at[(i % 2) * 3 + c], outs[c].at[pl.ds(base, _B)],
                    s_out)
                for c in range(3)]

    def start(cps):
        if not isinstance(cps, (list, tuple)):
            cps = [cps]
        for cp in cps:
            cp.start()

    def wait(cps):
        if not isinstance(cps, (list, tuple)):
            cps = [cps]
        for cp in cps:
            cp.wait()

    start(fid_copy(0))
    start(fid_copy(1))
    start(bary_copies(0))
    wait(fid_copy(0))
    start(vid_copies(0))
    wait(vid_copies(0))
    start(plane_copies(0))
    wait(fid_copy(1))
    start(vid_copies(1))
    start(fid_copy(2))

    def loop_body(i, carry):
        p = i % 2

        @pl.when(i + 1 < n_w)
        def _():
            wait(vid_copies(i + 1))
            start(plane_copies(i + 1))
            start(bary_copies(i + 1))

        @pl.when(i + 2 < n_w)
        def _():
            wait(fid_copy(i + 2))
            start(vid_copies(i + 2))

        @pl.when(i + 3 < n_w)
        def _():
            start(fid_copy(i + 3))

        wait(plane_copies(i))
        wait(bary_copies(i))

        @pl.when(i >= 2)
        def _():
            wait(out_copies(i - 2))

        gbase = p * 9
        bbase = p * 3
        for t in range(_B // _L):
            idx = lane + (t * _L)
            w0 = plsc.load_gather(bary_v, [_splat(bbase), idx])
            w1 = plsc.load_gather(bary_v, [_splat(bbase + 1), idx])
            w2 = plsc.load_gather(bary_v, [_splat(bbase + 2), idx])
            for c in range(3):
                g0 = plsc.load_gather(g_v, [_splat(gbase + c), idx])
                g1 = plsc.load_gather(g_v, [_splat(gbase + 3 + c), idx])
                g2 = plsc.load_gather(g_v, [_splat(gbase + 6 + c), idx])
                acc = w0 * g0 + w1 * g1 + w2 * g2
                plsc.store_scatter(out_v, [_splat(bbase + c), idx], acc)
        start(out_copies(i))
        return carry

    lax.fori_loop(0, n_w, loop_body, 0)
    wait(out_copies(n_w - 2))
    wait(out_copies(n_w - 1))


@jax.jit
def _sc_interp(fid, f0, f1, f2, vx, vy, vz, b0, b1, b2):
    mesh = plsc.VectorSubcoreMesh(core_axis_name="c", subcore_axis_name="s")
    out_plane = jax.ShapeDtypeStruct((_N_GAUSS,), jnp.float32)
    run = functools.partial(
        pl.kernel,
        mesh=mesh,
        compiler_params=pltpu.CompilerParams(needs_layout_passes=False),
        out_type=(out_plane, out_plane, out_plane),
        scratch_types=[
            pltpu.VMEM((2, _B), jnp.int32),
            pltpu.VMEM((9, _B), jnp.int32),
            pltpu.VMEM((18, _B), jnp.float32),
            pltpu.VMEM((6, _B), jnp.float32),
            pltpu.VMEM((6, _B), jnp.float32),
            pltpu.SemaphoreType.DMA,
            pltpu.SemaphoreType.DMA,
            pltpu.SemaphoreType.DMA,
            pltpu.SemaphoreType.DMA,
            pltpu.SemaphoreType.DMA,
        ],
    )(_body)
    return run(fid, f0, f1, f2, vx, vy, vz, b0, b1, b2)


def kernel(deformed_vertices, face, face_ids, face_bary):
    f0 = face[0]
    f1 = face[1]
    f2 = face[2]
    vx = deformed_vertices[:, 0]
    vy = deformed_vertices[:, 1]
    vz = deformed_vertices[:, 2]
    b0 = face_bary[:, 0]
    b1 = face_bary[:, 1]
    b2 = face_bary[:, 2]
    ox, oy, oz = _sc_interp(face_ids, f0, f1, f2, vx, vy, vz, b0, b1, b2)
    return jnp.stack([ox, oy, oz], axis=1)


# speedup vs baseline: 7.2357x; 1.0075x over previous
import functools

import jax
import jax.numpy as jnp
from jax import lax
from jax.experimental import pallas as pl
from jax.experimental.pallas import tpu as pltpu
from jax.experimental.pallas import tpu_sc as plsc

_N_GAUSS = 2_000_000
_B = 640
_W = 128
_S = _B // _W
_NCHUNKS = _N_GAUSS // _B
_NC = 2
_NS = 16
_NW = _NC * _NS
_L = 16


def _splat(v):
    return jnp.full((_L,), v, jnp.int32)


def _body(fid_hbm, f0_hbm, f1_hbm, f2_hbm, vx_hbm, vy_hbm, vz_hbm,
          b0_hbm, b1_hbm, b2_hbm,
          ox_hbm, oy_hbm, oz_hbm,
          fid_v, vid_v, g_v, bary_v, out_v,
          s_fid, s_bary, s_vid, s_pl, s_out):
    wid = lax.axis_index("s") * _NC + lax.axis_index("c")
    n_w = (_NCHUNKS - wid + _NW - 1) // _NW
    lane = lax.iota(jnp.int32, _L)
    planes = (vx_hbm, vy_hbm, vz_hbm)
    faces = (f0_hbm, f1_hbm, f2_hbm)
    barys = (b0_hbm, b1_hbm, b2_hbm)
    outs = (ox_hbm, oy_hbm, oz_hbm)

    def chunk(i):
        return wid + i * _NW

    def fid_copy(i):
        base = pl.multiple_of(chunk(i) * _B, _B)
        return pltpu.make_async_copy(
            fid_hbm.at[pl.ds(base, _B)], fid_v.at[i % 2], s_fid)

    def bary_copies(i):
        base = pl.multiple_of(chunk(i) * _B, _B)
        return [pltpu.make_async_copy(
                    barys[j].at[pl.ds(base, _B)],
                    bary_v.at[(i % 2) * 3 + j], s_bary)
                for j in range(3)]

    def vid_copies(i, s):
        sl = pl.ds(pl.multiple_of(s * _W, _W), _W)
        return [pltpu.make_async_copy(
                    faces[j].at[fid_v.at[i % 2, sl]],
                    vid_v.at[(i % 3) * 3 + j, sl], s_vid)
                for j in range(3)]

    def plane_copies(i, s):
        sl = pl.ds(pl.multiple_of(s * _W, _W), _W)
        return [pltpu.make_async_copy(
                    planes[c].at[vid_v.at[(i % 3) * 3 + j, sl]],
                    g_v.at[(i % 2) * 9 + 3 * j + c, sl], s_pl)
                for j in range(3) for c in range(3)]

    def out_copies(i):
        base = pl.multiple_of(chunk(i) * _B, _B)
        return [pltpu.make_async_copy(
                    out_v.at[(i % 2) * 3 + c], outs[c].at[pl.ds(base, _B)],
                    s_out)
                for c in range(3)]

    def start(cps):
        if not isinstance(cps, (list, tuple)):
            cps = [cps]
        for cp in cps:
            cp.start()

    def wait(cps):
        if not isinstance(cps, (list, tuple)):
            cps = [cps]
        for cp in cps:
            cp.wait()

    def start_sliced(mk, i):
        def body(s, c):
            start(mk(i, s))
            return c
        lax.fori_loop(0, _S, body, 0)

    def wait_sliced(mk, i):
        def body(s, c):
            wait(mk(i, s))
            return c
        lax.fori_loop(0, _S, body, 0)

    start(fid_copy(0))
    start(fid_copy(1))
    start(bary_copies(0))
    wait(fid_copy(0))
    start_sliced(vid_copies, 0)
    wait_sliced(vid_copies, 0)
    start_sliced(plane_copies, 0)
    wait(fid_copy(1))
    start_sliced(vid_copies, 1)
    start(fid_copy(2))

    def loop_body(i, carry):
        p = i % 2

        @pl.when(i + 1 < n_w)
        def _():
            wait_sliced(vid_copies, i + 1)
            start_sliced(plane_copies, i + 1)
            start(bary_copies(i + 1))

        @pl.when(i + 2 < n_w)
        def _():
            wait(fid_copy(i + 2))
            start_sliced(vid_copies, i + 2)

        @pl.when(i + 3 < n_w)
        def _():
            start(fid_copy(i + 3))

        wait_sliced(plane_copies, i)
        wait(bary_copies(i))

        @pl.when(i >= 2)
        def _():
            wait(out_copies(i - 2))

        gbase = p * 9
        bbase = p * 3

        def comp(t, c_):
            idx = lane + t * _L
            w0 = plsc.load_gather(bary_v, [_splat(bbase), idx])
            w1 = plsc.load_gather(bary_v, [_splat(bbase + 1), idx])
            w2 = plsc.load_gather(bary_v, [_splat(bbase + 2), idx])
            for c in range(3):
                g0 = plsc.load_gather(g_v, [_splat(gbase + c), idx])
                g1 = plsc.load_gather(g_v, [_splat(gbase + 3 + c), idx])
                g2 = plsc.load_gather(g_v, [_splat(gbase + 6 + c), idx])
                acc = w0 * g0 + w1 * g1 + w2 * g2
                plsc.store_scatter(out_v, [_splat(bbase + c), idx], acc)
            return c_

        lax.fori_loop(0, _B // _L, comp, 0)
        start(out_copies(i))
        return carry

    lax.fori_loop(0, n_w, loop_body, 0)
    wait(out_copies(n_w - 2))
    wait(out_copies(n_w - 1))


@jax.jit
def _sc_interp(fid, f0, f1, f2, vx, vy, vz, b0, b1, b2):
    mesh = plsc.VectorSubcoreMesh(core_axis_name="c", subcore_axis_name="s")
    out_plane = jax.ShapeDtypeStruct((_N_GAUSS,), jnp.float32)
    run = functools.partial(
        pl.kernel,
        mesh=mesh,
        compiler_params=pltpu.CompilerParams(needs_layout_passes=False),
        out_type=(out_plane, out_plane, out_plane),
        scratch_types=[
            pltpu.VMEM((2, _B), jnp.int32),
            pltpu.VMEM((9, _B), jnp.int32),
            pltpu.VMEM((18, _B), jnp.float32),
            pltpu.VMEM((6, _B), jnp.float32),
            pltpu.VMEM((6, _B), jnp.float32),
            pltpu.SemaphoreType.DMA,
            pltpu.SemaphoreType.DMA,
            pltpu.SemaphoreType.DMA,
            pltpu.SemaphoreType.DMA,
            pltpu.SemaphoreType.DMA,
        ],
    )(_body)
    return run(fid, f0, f1, f2, vx, vy, vz, b0, b1, b2)


def kernel(deformed_vertices, face, face_ids, face_bary):
    f0 = face[0]
    f1 = face[1]
    f2 = face[2]
    vx = deformed_vertices[:, 0]
    vy = deformed_vertices[:, 1]
    vz = deformed_vertices[:, 2]
    b0 = face_bary[:, 0]
    b1 = face_bary[:, 1]
    b2 = face_bary[:, 2]
    ox, oy, oz = _sc_interp(face_ids, f0, f1, f2, vx, vy, vz, b0, b1, b2)
    return jnp.stack([ox, oy, oz], axis=1)


# speedup vs baseline: 7.2669x; 1.0043x over previous
import functools

import jax
import jax.numpy as jnp
from jax import lax
from jax.experimental import pallas as pl
from jax.experimental.pallas import tpu as pltpu
from jax.experimental.pallas import tpu_sc as plsc

_N_GAUSS = 2_000_000
_B = 640
_W = 128
_S = _B // _W
_NCHUNKS = _N_GAUSS // _B
_NC = 2
_NS = 16
_NW = _NC * _NS
_L = 16


def _splat(v):
    return jnp.full((_L,), v, jnp.int32)


def _body(fid_hbm, f0_hbm, f1_hbm, f2_hbm, vx_hbm, vy_hbm, vz_hbm,
          b0_hbm, b1_hbm, b2_hbm,
          ox_hbm, oy_hbm, oz_hbm,
          fid_v, vid_v, g_v, bary_v, out_v,
          s_fid, s_bary, s_vid, s_pl, s_out):
    wid = lax.axis_index("s") * _NC + lax.axis_index("c")
    n_w = (_NCHUNKS - wid + _NW - 1) // _NW
    lane = lax.iota(jnp.int32, _L)
    planes = (vx_hbm, vy_hbm, vz_hbm)
    faces = (f0_hbm, f1_hbm, f2_hbm)
    barys = (b0_hbm, b1_hbm, b2_hbm)
    outs = (ox_hbm, oy_hbm, oz_hbm)

    def chunk(i):
        return wid + i * _NW

    def fid_copy(i):
        base = pl.multiple_of(chunk(i) * _B, _B)
        return pltpu.make_async_copy(
            fid_hbm.at[pl.ds(base, _B)], fid_v.at[i % 2], s_fid)

    def bary_copies(i):
        base = pl.multiple_of(chunk(i) * _B, _B)
        return [pltpu.make_async_copy(
                    barys[j].at[pl.ds(base, _B)],
                    bary_v.at[(i % 2) * 3 + j], s_bary)
                for j in range(3)]

    def vid_copies(i, s):
        sl = pl.ds(pl.multiple_of(s * _W, _W), _W)
        return [pltpu.make_async_copy(
                    faces[j].at[fid_v.at[i % 2, sl]],
                    vid_v.at[(i % 3) * 3 + j, sl], s_vid)
                for j in range(3)]

    def plane_copies(i, s):
        sl = pl.ds(pl.multiple_of(s * _W, _W), _W)
        return [pltpu.make_async_copy(
                    planes[c].at[vid_v.at[(i % 3) * 3 + j, sl]],
                    g_v.at[(i % 2) * 9 + 3 * j + c, sl], s_pl)
                for j in range(3) for c in range(3)]

    def out_copies(i):
        base = pl.multiple_of(chunk(i) * _B, _B)
        return [pltpu.make_async_copy(
                    out_v.at[(i % 2) * 3 + c], outs[c].at[pl.ds(base, _B)],
                    s_out)
                for c in range(3)]

    def start(cps):
        if not isinstance(cps, (list, tuple)):
            cps = [cps]
        for cp in cps:
            cp.start()

    def wait(cps):
        if not isinstance(cps, (list, tuple)):
            cps = [cps]
        for cp in cps:
            cp.wait()

    def start_sliced(mk, i):
        def body(s, c):
            start(mk(i, s))
            return c
        lax.fori_loop(0, _S, body, 0)

    def wait_sliced(mk, i):
        def body(s, c):
            wait(mk(i, s))
            return c
        lax.fori_loop(0, _S, body, 0)

    start(fid_copy(0))
    start(fid_copy(1))
    start(bary_copies(0))
    wait(fid_copy(0))
    start_sliced(vid_copies, 0)
    wait_sliced(vid_copies, 0)
    start_sliced(plane_copies, 0)
    wait(fid_copy(1))
    start_sliced(vid_copies, 1)
    start(fid_copy(2))

    def loop_body(i, carry):
        p = i % 2

        @pl.when(i + 1 < n_w)
        def _():
            wait_sliced(vid_copies, i + 1)
            start_sliced(plane_copies, i + 1)
            start(bary_copies(i + 1))

        @pl.when(i + 2 < n_w)
        def _():
            wait(fid_copy(i + 2))
            start_sliced(vid_copies, i + 2)

        @pl.when(i + 3 < n_w)
        def _():
            start(fid_copy(i + 3))

        wait_sliced(plane_copies, i)
        wait(bary_copies(i))

        @pl.when(i >= 2)
        def _():
            wait(out_copies(i - 2))

        def compute_static(pv):
            gb = pv * 9
            bb = pv * 3

            def comp(t, c_):
                sl = pl.ds(t * _L, _L)
                w0 = bary_v[bb, sl]
                w1 = bary_v[bb + 1, sl]
                w2 = bary_v[bb + 2, sl]
                for c in range(3):
                    acc = (w0 * g_v[gb + c, sl] + w1 * g_v[gb + 3 + c, sl]
                           + w2 * g_v[gb + 6 + c, sl])
                    out_v[bb + c, sl] = acc
                return c_

            lax.fori_loop(0, _B // _L, comp, 0)

        @pl.when(p == 0)
        def _():
            compute_static(0)

        @pl.when(p == 1)
        def _():
            compute_static(1)
        start(out_copies(i))
        return carry

    lax.fori_loop(0, n_w, loop_body, 0)
    wait(out_copies(n_w - 2))
    wait(out_copies(n_w - 1))


@jax.jit
def _sc_interp(fid, f0, f1, f2, vx, vy, vz, b0, b1, b2):
    mesh = plsc.VectorSubcoreMesh(core_axis_name="c", subcore_axis_name="s")
    out_plane = jax.ShapeDtypeStruct((_N_GAUSS,), jnp.float32)
    run = functools.partial(
        pl.kernel,
        mesh=mesh,
        compiler_params=pltpu.CompilerParams(needs_layout_passes=False),
        out_type=(out_plane, out_plane, out_plane),
        scratch_types=[
            pltpu.VMEM((2, _B), jnp.int32),
            pltpu.VMEM((9, _B), jnp.int32),
            pltpu.VMEM((18, _B), jnp.float32),
            pltpu.VMEM((6, _B), jnp.float32),
            pltpu.VMEM((6, _B), jnp.float32),
            pltpu.SemaphoreType.DMA,
            pltpu.SemaphoreType.DMA,
            pltpu.SemaphoreType.DMA,
            pltpu.SemaphoreType.DMA,
            pltpu.SemaphoreType.DMA,
        ],
    )(_body)
    return run(fid, f0, f1, f2, vx, vy, vz, b0, b1, b2)


def kernel(deformed_vertices, face, face_ids, face_bary):
    f0 = face[0]
    f1 = face[1]
    f2 = face[2]
    vx = deformed_vertices[:, 0]
    vy = deformed_vertices[:, 1]
    vz = deformed_vertices[:, 2]
    b0 = face_bary[:, 0]
    b1 = face_bary[:, 1]
    b2 = face_bary[:, 2]
    ox, oy, oz = _sc_interp(face_ids, f0, f1, f2, vx, vy, vz, b0, b1, b2)
    return jnp.stack([ox, oy, oz], axis=1)
